# Initial kernel scaffold; baseline (speedup 1.0000x reference)
import jax
import jax.numpy as jnp
from jax.experimental import pallas as pl


def kernel(X, E, emb_nodes, emb_edges, edge_index, att_W1, att_b1, att_W2, att_b2, att_W3, att_b3, msg_W1, msg_b1, msg_W2, msg_b2):
    raise NotImplementedError("write your pallas kernel here")


# speedup vs baseline: 1.8428x; 1.8428x over previous
import functools

import jax
import jax.numpy as jnp
from jax import lax
from jax.experimental import pallas as pl
from jax.experimental.pallas import tpu as pltpu
from jax.experimental.pallas import tpu_sc as plsc

NC = 2
NS = 16
NW = NC * NS
CHUNK = 128


def _sc_mesh():
    return plsc.VectorSubcoreMesh(
        core_axis_name="c", subcore_axis_name="s", num_cores=NC, num_subcores=NS
    )


def _make_gather(ne_pad, n, d, nch):
    @functools.partial(
        pl.kernel,
        mesh=_sc_mesh(),
        out_type=[
            jax.ShapeDtypeStruct((ne_pad, d), jnp.float32),
            jax.ShapeDtypeStruct((ne_pad, d), jnp.float32),
        ],
        scratch_types=[
            pltpu.VMEM((CHUNK,), jnp.int32),
            pltpu.VMEM((CHUNK,), jnp.int32),
            pltpu.VMEM((CHUNK, d), jnp.float32),
            pltpu.VMEM((CHUNK, d), jnp.float32),
            pltpu.SemaphoreType.DMA,
            pltpu.SemaphoreType.DMA,
        ],
    )
    def gather_k(x_hbm, src_hbm, dst_hbm, xs_out, xd_out, sidx, didx, srows, drows, s1, s2):
        cid = lax.axis_index("c")
        sid = lax.axis_index("s")
        wid = sid * NC + cid

        def body(c, _):
            base = (wid * nch + c) * CHUNK
            pltpu.sync_copy(src_hbm.at[pl.ds(base, CHUNK)], sidx)
            pltpu.sync_copy(dst_hbm.at[pl.ds(base, CHUNK)], didx)
            cp1 = pltpu.async_copy(x_hbm.at[sidx], srows, s1)
            cp2 = pltpu.async_copy(x_hbm.at[didx], drows, s2)
            cp1.wait()
            cp2.wait()
            pltpu.sync_copy(srows, xs_out.at[pl.ds(base, CHUNK)])
            pltpu.sync_copy(drows, xd_out.at[pl.ds(base, CHUNK)])
            return 0

        lax.fori_loop(0, nch, body, 0)

    return gather_k


def _make_scatter(ne_pad, n_pad, d, nch):
    rows_per_tile = n_pad // NS

    @functools.partial(
        pl.kernel,
        mesh=_sc_mesh(),
        out_type=jax.ShapeDtypeStruct((NC, n_pad, d), jnp.float32),
        scratch_types=[
            pltpu.VMEM_SHARED((n_pad, d), jnp.float32),
            pltpu.VMEM((CHUNK,), jnp.int32),
            pltpu.VMEM((CHUNK, d), jnp.float32),
        ],
    )
    def scatter_k(m_hbm, dst_hbm, zero_hbm, acc_out, acc, didx, mrows):
        cid = lax.axis_index("c")
        sid = lax.axis_index("s")
        wid = sid * NC + cid

        r0 = sid * rows_per_tile
        pltpu.sync_copy(zero_hbm.at[pl.ds(r0, rows_per_tile)], acc.at[pl.ds(r0, rows_per_tile)])
        plsc.subcore_barrier()

        def body(c, _):
            base = (wid * nch + c) * CHUNK
            pltpu.sync_copy(dst_hbm.at[pl.ds(base, CHUNK)], didx)
            pltpu.sync_copy(m_hbm.at[pl.ds(base, CHUNK)], mrows)
            pltpu.sync_copy(mrows, acc.at[didx], add=True)
            return 0

        lax.fori_loop(0, nch, body, 0)
        plsc.subcore_barrier()
        pltpu.sync_copy(acc.at[pl.ds(r0, rows_per_tile)], acc_out.at[cid, pl.ds(r0, rows_per_tile)])

    return scatter_k


def _mlp_body(ne, be, xs_ref, xd_ref, e_ref,
              aw1s_ref, aw1d_ref, aw1e_ref, ab1_ref, aw2_ref, ab2_ref, aw3_ref, ab3_ref,
              mw1s_ref, mw1d_ref, mw1e_ref, mb1_ref, mw2_ref, mb2_ref, out_ref):
    xs = xs_ref[...]
    xd = xd_ref[...]
    e = e_ref[...]
    dot = functools.partial(jnp.dot, preferred_element_type=jnp.float32)
    t = dot(xs, aw1s_ref[...]) + dot(xd, aw1d_ref[...]) + dot(e, aw1e_ref[...]) + ab1_ref[...]
    t = jnp.maximum(t, 0.0)
    t = jnp.maximum(dot(t, aw2_ref[...]) + ab2_ref[...], 0.0)
    a = jnp.sum(t * aw3_ref[...], axis=-1, keepdims=True) + ab3_ref[...]
    h = dot(xs, mw1s_ref[...]) + dot(xd, mw1d_ref[...]) + dot(e, mw1e_ref[...]) + mb1_ref[...]
    h = jnp.maximum(h, 0.0)
    m = dot(h, mw2_ref[...]) + mb2_ref[...]
    msg = jax.nn.sigmoid(a) * m
    row = pl.program_id(0) * be + lax.broadcasted_iota(jnp.int32, msg.shape, 0)
    out_ref[...] = jnp.where(row < ne, msg, 0.0)


def _combine_body(x_ref, a_ref, out_ref):
    out_ref[...] = x_ref[...] + a_ref[0] + a_ref[1]


def kernel(X, E, emb_nodes, emb_edges, edge_index,
           att_W1, att_b1, att_W2, att_b2, att_W3, att_b3,
           msg_W1, msg_b1, msg_W2, msg_b2):
    n, d = X.shape
    ne = E.shape[0]
    nch = -(-ne // (NW * CHUNK))
    ne_pad = nch * NW * CHUNK
    pad = ne_pad - ne

    src = jnp.concatenate([edge_index[0], jnp.zeros((pad,), jnp.int32)])
    dst = jnp.concatenate([edge_index[1], jnp.zeros((pad,), jnp.int32)])
    e_pad = jnp.concatenate([E, jnp.zeros((pad, d), jnp.float32)], axis=0)

    xs, xd = _make_gather(ne_pad, n, d, nch)(X, src, dst)

    be = 2048
    grid = ne_pad // be

    def full(shape):
        return pl.BlockSpec(shape, lambda i: tuple(0 for _ in shape))

    m_arr = pl.pallas_call(
        functools.partial(_mlp_body, ne, be),
        grid=(grid,),
        in_specs=[
            pl.BlockSpec((be, d), lambda i: (i, 0)),
            pl.BlockSpec((be, d), lambda i: (i, 0)),
            pl.BlockSpec((be, d), lambda i: (i, 0)),
            full((d, 96)), full((d, 96)), full((d, 96)), full((1, 96)),
            full((96, 48)), full((1, 48)), full((1, 48)), full((1, 1)),
            full((d, 256)), full((d, 256)), full((d, 256)), full((1, 256)),
            full((256, d)), full((1, d)),
        ],
        out_specs=pl.BlockSpec((be, d), lambda i: (i, 0)),
        out_shape=jax.ShapeDtypeStruct((ne_pad, d), jnp.float32),
        compiler_params=pltpu.CompilerParams(
            dimension_semantics=("arbitrary",),
        ),
    )(
        xs, xd, e_pad,
        att_W1[:d], att_W1[d:2 * d], att_W1[2 * d:], att_b1[None, :],
        att_W2, att_b2[None, :], att_W3.T, att_b3[None, :],
        msg_W1[:d], msg_W1[d:2 * d], msg_W1[2 * d:], msg_b1[None, :],
        msg_W2, msg_b2[None, :],
    )

    n_pad = NS * 8 * (-(-n // (NS * 8)))
    zeros_nd = jnp.zeros((n_pad, d), jnp.float32)
    accs = _make_scatter(ne_pad, n_pad, d, nch)(m_arr, dst, zeros_nd)

    bn = 2000
    x_out = pl.pallas_call(
        _combine_body,
        grid=(n // bn,),
        in_specs=[
            pl.BlockSpec((bn, d), lambda i: (i, 0)),
            pl.BlockSpec((NC, bn, d), lambda i: (0, i, 0)),
        ],
        out_specs=pl.BlockSpec((bn, d), lambda i: (i, 0)),
        out_shape=jax.ShapeDtypeStruct((n, d), jnp.float32),
    )(X, accs)

    return (x_out, E)
